# baseline (device time: 118434 ns/iter reference)
import jax
import jax.numpy as jnp
from jax import lax
from jax.experimental import pallas as pl
from jax.experimental.pallas import tpu as pltpu

N_DEV = 8
B = 2
SQ = 512
SKV = 512
HKV = SKV // 2
E = 768
H = 8
DH = 64
HD = H * DH
BH = B * H
NEG = -1e9
QSCALE = 0.125 * 1.4426950408889634


def kernel(x, Wq, K_ext, V_ext, Wo):
    def body(x_ref, wq_ref, k_ref, v_ref, wo_ref, out_ref,
             qh, kvfull, khead, vhead, acc, lrow,
             sA, rA, sB, rB):
        my = lax.axis_index("i")

        def pr(t):
            return jnp.where(t < 4, t, 11 - t)

        r = pr(my)
        right = pr(lax.rem(r + 1, N_DEV))
        left = pr(lax.rem(r - 1 + N_DEV, N_DEV))

        barrier = pltpu.get_barrier_semaphore()
        for nbr in (left, right):
            pl.semaphore_signal(barrier, inc=1, device_id=(nbr,),
                                device_id_type=pl.DeviceIdType.MESH)

        kvfull[my, 0] = k_ref[...].astype(jnp.bfloat16).reshape(B, SKV, HD)
        kvfull[my, 1] = v_ref[...].astype(jnp.bfloat16).reshape(B, SKV, HD)

        wq = wq_ref[...].astype(jnp.bfloat16)
        for b in range(B):
            xb = x_ref[b].astype(jnp.bfloat16)
            qb = lax.dot_general(xb, wq, (((1,), (0,)), ((), ())),
                                 preferred_element_type=jnp.float32)
            qb = (qb * QSCALE).astype(jnp.bfloat16)
            for h in range(H):
                qh[b * H + h] = qb[:, h * DH:(h + 1) * DH]

        lrow[...] = jnp.zeros((BH, 1, SQ), jnp.float32)
        acc[...] = jnp.zeros((BH, DH, SQ), jnp.float32)

        def unpack(org_a, org_b):
            for b in range(B):
                ka = kvfull[org_a, 0, b]
                va = kvfull[org_a, 1, b]
                kb_ = kvfull[org_b, 0, b]
                vb_ = kvfull[org_b, 1, b]
                for h in range(H):
                    sl = slice(h * DH, (h + 1) * DH)
                    khead[b * H + h, 0:HKV] = ka[0:HKV, sl]
                    vhead[b * H + h, 0:HKV] = va[0:HKV, sl]
                    khead[b * H + h, HKV:SKV] = kb_[HKV:SKV, sl]
                    vhead[b * H + h, HKV:SKV] = vb_[HKV:SKV, sl]

        unpack(my, my)

        pl.semaphore_wait(barrier, 2)

        qb2 = my * H + lax.broadcasted_iota(jnp.int32, (SKV, SQ), 1) // 64
        row_i = lax.broadcasted_iota(jnp.int32, (SKV, SQ), 0)

        def fold(org_a, org_b):
            org_row = jnp.where(row_i < HKV, org_a, org_b)
            kb2 = org_row * H + row_i // 64
            keep = (qb2 == kb2) | (kb2 == 0) | (lax.rem(qb2 + kb2, 3) == 0)
            biasT = jnp.where(keep, 0.0, NEG).astype(jnp.float32)

            def bh_body(bh, carry):
                q = qh[bh]
                k = khead[bh]
                sT = lax.dot_general(k, q, (((1,), (1,)), ((), ())),
                                     preferred_element_type=jnp.float32)
                p = jnp.exp2(sT + biasT)
                lrow[bh] = lrow[bh] + jnp.sum(p, axis=0, keepdims=True)
                pv = lax.dot_general(vhead[bh], p.astype(jnp.bfloat16),
                                     (((0,), (0,)), ((), ())),
                                     preferred_element_type=jnp.float32)
                acc[bh] = acc[bh] + pv
                return carry

            lax.fori_loop(0, BH, bh_body, 0)

        NQ = 4
        QKV = HKV // NQ

        def sub_rdma(hp, qi, org_a, org_b, clockwise):
            if clockwise:
                org, off, dev, ss, rs = org_a, qi * QKV, right, sA, rA
            else:
                org, off, dev, ss, rs = org_b, HKV + qi * QKV, left, sB, rB
            return pltpu.make_async_remote_copy(
                src_ref=kvfull.at[org, :, :, pl.ds(off, QKV), :],
                dst_ref=kvfull.at[org, :, :, pl.ds(off, QKV), :],
                send_sem=ss.at[hp, qi], recv_sem=rs.at[hp, qi],
                device_id=(dev,), device_id_type=pl.DeviceIdType.MESH)

        orgs = []
        for hp in range(N_DEV):
            orgs.append((pr(lax.rem(r - hp + 2 * N_DEV, N_DEV)),
                         pr(lax.rem(r + hp, N_DEV))))

        for qi in range(NQ):
            sub_rdma(0, qi, *orgs[0], True).start()
            sub_rdma(0, qi, *orgs[0], False).start()
        fold(*orgs[0])
        for hp in range(1, N_DEV - 1):
            for qi in range(NQ):
                sub_rdma(hp - 1, qi, *orgs[hp - 1], True).wait_recv()
                sub_rdma(hp - 1, qi, *orgs[hp - 1], False).wait_recv()
                sub_rdma(hp, qi, *orgs[hp], True).start()
                sub_rdma(hp, qi, *orgs[hp], False).start()
            unpack(*orgs[hp])
            fold(*orgs[hp])
        for qi in range(NQ):
            sub_rdma(N_DEV - 2, qi, *orgs[N_DEV - 2], True).wait_recv()
            sub_rdma(N_DEV - 2, qi, *orgs[N_DEV - 2], False).wait_recv()
        unpack(*orgs[N_DEV - 1])
        fold(*orgs[N_DEV - 1])

        for hp in range(N_DEV - 1):
            for qi in range(NQ):
                sub_rdma(hp, qi, *orgs[hp], True).wait_send()
                sub_rdma(hp, qi, *orgs[hp], False).wait_send()

        wo = wo_ref[...].astype(jnp.bfloat16)
        for b in range(B):
            parts = []
            for h in range(H):
                bh = b * H + h
                parts.append(acc[bh] / lrow[bh])
            ctxT = jnp.concatenate(parts, axis=0).astype(jnp.bfloat16)
            out_ref[b] = lax.dot_general(ctxT, wo, (((0,), (0,)), ((), ())),
                                         preferred_element_type=jnp.float32)

    return pl.pallas_call(
        body,
        out_shape=jax.ShapeDtypeStruct((B, SQ, E), jnp.float32),
        in_specs=[pl.BlockSpec(memory_space=pltpu.VMEM)] * 5,
        out_specs=pl.BlockSpec(memory_space=pltpu.VMEM),
        scratch_shapes=[
            pltpu.VMEM((BH, SQ, DH), jnp.bfloat16),
            pltpu.VMEM((N_DEV, 2, B, SKV, HD), jnp.bfloat16),
            pltpu.VMEM((BH, SKV, DH), jnp.bfloat16),
            pltpu.VMEM((BH, SKV, DH), jnp.bfloat16),
            pltpu.VMEM((BH, DH, SQ), jnp.float32),
            pltpu.VMEM((BH, 1, SQ), jnp.float32),
        ] + [pltpu.SemaphoreType.DMA((N_DEV - 1, 4))] * 4,
        compiler_params=pltpu.CompilerParams(
            collective_id=0, vmem_limit_bytes=100 * 1024 * 1024),
    )(x, Wq, K_ext, V_ext, Wo)


# device time: 116096 ns/iter; 1.0201x vs baseline; 1.0201x over previous
import jax
import jax.numpy as jnp
from jax import lax
from jax.experimental import pallas as pl
from jax.experimental.pallas import tpu as pltpu

N_DEV = 8
B = 2
SQ = 512
SKV = 512
HKV = SKV // 2
E = 768
H = 8
DH = 64
HD = H * DH
BH = B * H
NEG = -1e9
QSCALE = 0.125 * 1.4426950408889634


def kernel(x, Wq, K_ext, V_ext, Wo):
    def body(x_ref, wq_ref, k_ref, v_ref, wo_ref, out_ref,
             qh, kvfull, khead, vhead, acc, lrow,
             sA, rA, sB, rB):
        my = lax.axis_index("i")

        def pr(t):
            return jnp.where(t < 4, t, 11 - t)

        r = pr(my)
        right = pr(lax.rem(r + 1, N_DEV))
        left = pr(lax.rem(r - 1 + N_DEV, N_DEV))

        barrier = pltpu.get_barrier_semaphore()
        for nbr in (left, right):
            pl.semaphore_signal(barrier, inc=1, device_id=(nbr,),
                                device_id_type=pl.DeviceIdType.MESH)

        kvfull[my, 0] = k_ref[...].astype(jnp.bfloat16).reshape(B, SKV, HD)
        kvfull[my, 1] = v_ref[...].astype(jnp.bfloat16).reshape(B, SKV, HD)

        def local_prep():
            wq = wq_ref[...].astype(jnp.bfloat16)
            for b in range(B):
                xb = x_ref[b].astype(jnp.bfloat16)
                qb = lax.dot_general(xb, wq, (((1,), (0,)), ((), ())),
                                     preferred_element_type=jnp.float32)
                qb = (qb * QSCALE).astype(jnp.bfloat16)
                for h in range(H):
                    qh[b * H + h] = qb[:, h * DH:(h + 1) * DH]
            lrow[...] = jnp.zeros((BH, 1, SQ), jnp.float32)
            acc[...] = jnp.zeros((BH, DH, SQ), jnp.float32)

        def unpack(org_a, org_b):
            for b in range(B):
                ka = kvfull[org_a, 0, b]
                va = kvfull[org_a, 1, b]
                kb_ = kvfull[org_b, 0, b]
                vb_ = kvfull[org_b, 1, b]
                for h in range(H):
                    sl = slice(h * DH, (h + 1) * DH)
                    khead[b * H + h, 0:HKV] = ka[0:HKV, sl]
                    vhead[b * H + h, 0:HKV] = va[0:HKV, sl]
                    khead[b * H + h, HKV:SKV] = kb_[HKV:SKV, sl]
                    vhead[b * H + h, HKV:SKV] = vb_[HKV:SKV, sl]

        pl.semaphore_wait(barrier, 2)

        qb2 = my * H + lax.broadcasted_iota(jnp.int32, (SKV, SQ), 1) // 64
        row_i = lax.broadcasted_iota(jnp.int32, (SKV, SQ), 0)

        def fold(org_a, org_b):
            org_row = jnp.where(row_i < HKV, org_a, org_b)
            kb2 = org_row * H + row_i // 64
            keep = (qb2 == kb2) | (kb2 == 0) | (lax.rem(qb2 + kb2, 3) == 0)
            biasT = jnp.where(keep, 0.0, NEG).astype(jnp.float32)

            def bh_body(bh, carry):
                q = qh[bh]
                k = khead[bh]
                sT = lax.dot_general(k, q, (((1,), (1,)), ((), ())),
                                     preferred_element_type=jnp.float32)
                p = jnp.exp2(sT + biasT)
                lrow[bh] = lrow[bh] + jnp.sum(p, axis=0, keepdims=True)
                pv = lax.dot_general(vhead[bh], p.astype(jnp.bfloat16),
                                     (((0,), (0,)), ((), ())),
                                     preferred_element_type=jnp.float32)
                acc[bh] = acc[bh] + pv
                return carry

            lax.fori_loop(0, BH, bh_body, 0)

        NQ = 4
        QKV = HKV // NQ

        def sub_rdma(hp, qi, org_a, org_b, clockwise):
            if clockwise:
                org, off, dev, ss, rs = org_a, qi * QKV, right, sA, rA
            else:
                org, off, dev, ss, rs = org_b, HKV + qi * QKV, left, sB, rB
            return pltpu.make_async_remote_copy(
                src_ref=kvfull.at[org, :, :, pl.ds(off, QKV), :],
                dst_ref=kvfull.at[org, :, :, pl.ds(off, QKV), :],
                send_sem=ss.at[hp, qi], recv_sem=rs.at[hp, qi],
                device_id=(dev,), device_id_type=pl.DeviceIdType.MESH)

        orgs = []
        for hp in range(N_DEV):
            orgs.append((pr(lax.rem(r - hp + 2 * N_DEV, N_DEV)),
                         pr(lax.rem(r + hp, N_DEV))))

        for qi in range(NQ):
            sub_rdma(0, qi, *orgs[0], True).start()
            sub_rdma(0, qi, *orgs[0], False).start()
        local_prep()
        unpack(my, my)
        fold(*orgs[0])
        for hp in range(1, N_DEV - 1):
            for qi in range(NQ):
                sub_rdma(hp - 1, qi, *orgs[hp - 1], True).wait_recv()
                sub_rdma(hp - 1, qi, *orgs[hp - 1], False).wait_recv()
                sub_rdma(hp, qi, *orgs[hp], True).start()
                sub_rdma(hp, qi, *orgs[hp], False).start()
            unpack(*orgs[hp])
            fold(*orgs[hp])
        for qi in range(NQ):
            sub_rdma(N_DEV - 2, qi, *orgs[N_DEV - 2], True).wait_recv()
            sub_rdma(N_DEV - 2, qi, *orgs[N_DEV - 2], False).wait_recv()
        unpack(*orgs[N_DEV - 1])
        fold(*orgs[N_DEV - 1])

        for hp in range(N_DEV - 1):
            for qi in range(NQ):
                sub_rdma(hp, qi, *orgs[hp], True).wait_send()
                sub_rdma(hp, qi, *orgs[hp], False).wait_send()

        wo = wo_ref[...].astype(jnp.bfloat16)
        for b in range(B):
            parts = []
            for h in range(H):
                bh = b * H + h
                parts.append(acc[bh] / lrow[bh])
            ctxT = jnp.concatenate(parts, axis=0).astype(jnp.bfloat16)
            out_ref[b] = lax.dot_general(ctxT, wo, (((0,), (0,)), ((), ())),
                                         preferred_element_type=jnp.float32)

    return pl.pallas_call(
        body,
        out_shape=jax.ShapeDtypeStruct((B, SQ, E), jnp.float32),
        in_specs=[pl.BlockSpec(memory_space=pltpu.VMEM)] * 5,
        out_specs=pl.BlockSpec(memory_space=pltpu.VMEM),
        scratch_shapes=[
            pltpu.VMEM((BH, SQ, DH), jnp.bfloat16),
            pltpu.VMEM((N_DEV, 2, B, SKV, HD), jnp.bfloat16),
            pltpu.VMEM((BH, SKV, DH), jnp.bfloat16),
            pltpu.VMEM((BH, SKV, DH), jnp.bfloat16),
            pltpu.VMEM((BH, DH, SQ), jnp.float32),
            pltpu.VMEM((BH, 1, SQ), jnp.float32),
        ] + [pltpu.SemaphoreType.DMA((N_DEV - 1, 4))] * 4,
        compiler_params=pltpu.CompilerParams(
            collective_id=0, vmem_limit_bytes=100 * 1024 * 1024),
    )(x, Wq, K_ext, V_ext, Wo)
